# baseline (speedup 1.0000x reference)
import functools

import jax
import jax.numpy as jnp
from jax import lax
from jax.experimental import pallas as pl
from jax.experimental.pallas import tpu as pltpu
from jax.experimental.pallas import tpu_sc as plsc

_NUM_MOVIES = 100000
_D = 64
_NU = 512
_K = 200
_BPW = 8
_NWORK = _K // _BPW
_NS = 16
_TCB = 20
_KPAD = _K
_NB = _NU // 8


def _sc_body(q_hbm, idx_hbm, tabT_hbm, out_e_hbm,
             idx_v, slab_v, cat_v, sem):
    wid = lax.axis_index("c") * _NS + lax.axis_index("s")

    @pl.when(wid < _NWORK)
    def _():
        base = wid * _BPW
        pltpu.sync_copy(idx_hbm.at[pl.ds(base, _BPW)], idx_v.at[pl.ds(0, _BPW)])
        pltpu.sync_copy(q_hbm, idx_v.at[pl.ds(_BPW, 1)])
        ivec = idx_v[...]

        copies = []
        for j in range(_BPW + 1):
            m = ivec[j]
            start = pl.multiple_of((m // 128) * 128, 128)
            copies.append(pltpu.make_async_copy(
                tabT_hbm.at[:, pl.ds(start, 128)], slab_v.at[j], sem))
        for c in copies:
            c.start()
        for c in copies:
            c.wait()

        lane = lax.iota(jnp.int32, 16)
        offs = ivec % 128
        qoff = jnp.full((16,), offs[_BPW], jnp.int32)
        for c in range(_D // 16):
            qchunk = plsc.load_gather(
                slab_v, [jnp.full((16,), _BPW, jnp.int32), lane + c * 16,
                         qoff])
            for j in range(_BPW):
                joff = jnp.full((16,), offs[j], jnp.int32)
                cat_v[j, pl.ds(c * 16, 16)] = plsc.load_gather(
                    slab_v, [jnp.full((16,), j, jnp.int32), lane + c * 16,
                             joff])
                cat_v[j, pl.ds(_D + c * 16, 16)] = qchunk

        pltpu.sync_copy(cat_v, out_e_hbm.at[pl.ds(base, _BPW)])


@functools.lru_cache(maxsize=None)
def _build_sc_kernel():
    return pl.kernel(
        _sc_body,
        out_type=jax.ShapeDtypeStruct((_K, 2 * _D), jnp.float32),
        mesh=plsc.VectorSubcoreMesh(core_axis_name="c", subcore_axis_name="s"),
        compiler_params=pltpu.CompilerParams(needs_layout_passes=False),
        scratch_types=[
            pltpu.VMEM((16,), jnp.int32),
            pltpu.VMEM((_BPW + 1, _D, 128), jnp.float32),
            pltpu.VMEM((_BPW, 2 * _D), jnp.float32),
            pltpu.SemaphoreType.DMA,
        ],
    )


def _tc_iou_body(sidx_ref, qidx_ref, qblk_ref, *args):
    sblk_refs = args[:_TCB]
    out_ref = args[_TCB]
    g = pl.program_id(0)
    nb = _TCB + 1

    all_b = jnp.concatenate(
        [qblk_ref[...].astype(jnp.int8)]
        + [sblk_refs[j][...].astype(jnp.int8) for j in range(_TCB)],
        axis=0)
    cols = [jnp.full((1, 1), qidx_ref[0] % 32, jnp.int32)]
    for j in range(_TCB):
        cols.append(jnp.full((1, 1),
                             32 * (j + 1) + sidx_ref[g * _TCB + j] % 32,
                             jnp.int32))
    colv = jnp.concatenate(cols, axis=0)
    oh = (lax.broadcasted_iota(jnp.int32, (nb, 32 * nb), 1)
          == colv).astype(jnp.int8)
    sel = jax.lax.dot_general(
        oh, all_b, (((1,), (0,)), ((), ())),
        preferred_element_type=jnp.int32) & 0xFF
    qrow = sel[0:1, :]
    inter_pc = lax.population_count(sel & qrow).astype(jnp.float32)
    row_pc = lax.population_count(sel).astype(jnp.float32)
    ones = jnp.ones((1, _NB), jnp.float32)
    inter_all = jax.lax.dot_general(
        ones, inter_pc, (((1,), (1,)), ((), ())),
        preferred_element_type=jnp.float32)
    len_all = jax.lax.dot_general(
        ones, row_pc, (((1,), (1,)), ((), ())),
        preferred_element_type=jnp.float32)
    inter = inter_all[0:1, 1:]
    s_len = len_all[0:1, 1:]
    q_len = len_all[0:1, 0:1]
    union = q_len + s_len - inter
    out_ref[pl.ds(g, 1), :] = jnp.where(
        union > 0, inter / jnp.maximum(union, 1.0), 0.0)


@functools.lru_cache(maxsize=None)
def _build_tc_kernel():
    def sblk_spec(j):
        return pl.BlockSpec(
            (32, _NB), lambda g, sidx, qidx, j=j: (sidx[g * _TCB + j] // 32, 0))

    return pl.pallas_call(
        _tc_iou_body,
        grid_spec=pltpu.PrefetchScalarGridSpec(
            num_scalar_prefetch=2,
            grid=(_KPAD // _TCB,),
            in_specs=[
                pl.BlockSpec((32, _NB), lambda g, sidx, qidx: (qidx[0] // 32, 0)),
            ] + [sblk_spec(j) for j in range(_TCB)],
            out_specs=pl.BlockSpec((_KPAD // _TCB, _TCB),
                                   lambda g, sidx, qidx: (0, 0)),
        ),
        out_shape=jax.ShapeDtypeStruct((_KPAD // _TCB, _TCB), jnp.float32),
    )


def kernel(query, support_set, embed_table, user_sets):
    cat_embeds = _build_sc_kernel()(query, support_set, embed_table.T)
    ub = jnp.packbits(user_sets, axis=1)
    iou = _build_tc_kernel()(
        support_set, query, *([ub] * (_TCB + 1)))
    return cat_embeds, iou.reshape(_K, 1)


# speedup vs baseline: 6.2493x; 6.2493x over previous
import functools

import jax
import jax.numpy as jnp
from jax import lax
from jax.experimental import pallas as pl
from jax.experimental.pallas import tpu as pltpu
from jax.experimental.pallas import tpu_sc as plsc

_NUM_MOVIES = 100000
_D = 64
_NU = 512
_K = 200
_BPW = 8
_NWORK = _K // _BPW
_NS = 16
_TCB = 20
_KPAD = _K


def _sc_body(q_hbm, idx_hbm, tabT_hbm, out_e_hbm,
             idx_v, slab_v, cat_v, sem):
    wid = lax.axis_index("c") * _NS + lax.axis_index("s")

    @pl.when(wid < _NWORK)
    def _():
        base = wid * _BPW
        pltpu.sync_copy(idx_hbm.at[pl.ds(base, _BPW)], idx_v.at[pl.ds(0, _BPW)])
        pltpu.sync_copy(q_hbm, idx_v.at[pl.ds(_BPW, 1)])
        ivec = idx_v[...]

        copies = []
        for j in range(_BPW + 1):
            m = ivec[j]
            start = pl.multiple_of((m // 128) * 128, 128)
            copies.append(pltpu.make_async_copy(
                tabT_hbm.at[:, pl.ds(start, 128)], slab_v.at[j], sem))
        for c in copies:
            c.start()
        for c in copies:
            c.wait()

        lane = lax.iota(jnp.int32, 16)
        offs = ivec % 128
        qoff = jnp.full((16,), offs[_BPW], jnp.int32)
        for c in range(_D // 16):
            qchunk = plsc.load_gather(
                slab_v, [jnp.full((16,), _BPW, jnp.int32), lane + c * 16,
                         qoff])
            for j in range(_BPW):
                joff = jnp.full((16,), offs[j], jnp.int32)
                cat_v[j, pl.ds(c * 16, 16)] = plsc.load_gather(
                    slab_v, [jnp.full((16,), j, jnp.int32), lane + c * 16,
                             joff])
                cat_v[j, pl.ds(_D + c * 16, 16)] = qchunk

        pltpu.sync_copy(cat_v, out_e_hbm.at[pl.ds(base, _BPW)])


@functools.lru_cache(maxsize=None)
def _build_sc_kernel():
    return pl.kernel(
        _sc_body,
        out_type=jax.ShapeDtypeStruct((_K, 2 * _D), jnp.float32),
        mesh=plsc.VectorSubcoreMesh(core_axis_name="c", subcore_axis_name="s"),
        compiler_params=pltpu.CompilerParams(needs_layout_passes=False),
        scratch_types=[
            pltpu.VMEM((16,), jnp.int32),
            pltpu.VMEM((_BPW + 1, _D, 128), jnp.float32),
            pltpu.VMEM((_BPW, 2 * _D), jnp.float32),
            pltpu.SemaphoreType.DMA,
        ],
    )


def _tc_iou_body(sidx_ref, qidx_ref, qblk_ref, *args):
    sblk_refs = args[:_TCB]
    out_ref = args[_TCB]
    g = pl.program_id(0)
    nb = _TCB + 1

    oh_q = (lax.broadcasted_iota(jnp.int32, (1, 32), 1)
            == qidx_ref[0] % 32).astype(jnp.int8)
    qrow = jax.lax.dot_general(
        oh_q, qblk_ref[...].astype(jnp.int8), (((1,), (0,)), ((), ())),
        preferred_element_type=jnp.int32).astype(jnp.int8)
    v_mat = jnp.concatenate(
        [qrow, jnp.ones((1, _NU), jnp.int8)], axis=0)

    s_all = jnp.concatenate(
        [qblk_ref[...].astype(jnp.int8)]
        + [sblk_refs[j][...].astype(jnp.int8) for j in range(_TCB)],
        axis=0)
    p_all = jax.lax.dot_general(
        v_mat, s_all, (((1,), (1,)), ((), ())),
        preferred_element_type=jnp.int32).astype(jnp.float32)

    cols = [jnp.full((1, 1), qidx_ref[0] % 32, jnp.int32)]
    for j in range(_TCB):
        cols.append(jnp.full((1, 1),
                             32 * (j + 1) + sidx_ref[g * _TCB + j] % 32,
                             jnp.int32))
    colv = jnp.concatenate(cols, axis=0)
    oh = (lax.broadcasted_iota(jnp.int32, (nb, 32 * nb), 1)
          == colv).astype(jnp.float32)
    r = jax.lax.dot_general(
        p_all, oh, (((1,), (1,)), ((), ())),
        preferred_element_type=jnp.float32)
    inter = r[0:1, 1:]
    s_len = r[1:2, 1:]
    q_len = r[1:2, 0:1]
    union = q_len + s_len - inter
    out_ref[pl.ds(g, 1), :] = jnp.where(
        union > 0, inter / jnp.maximum(union, 1.0), 0.0)


@functools.lru_cache(maxsize=None)
def _build_tc_kernel():
    def sblk_spec(j):
        return pl.BlockSpec(
            (32, _NU), lambda g, sidx, qidx, j=j: (sidx[g * _TCB + j] // 32, 0))

    return pl.pallas_call(
        _tc_iou_body,
        grid_spec=pltpu.PrefetchScalarGridSpec(
            num_scalar_prefetch=2,
            grid=(_KPAD // _TCB,),
            in_specs=[
                pl.BlockSpec((32, _NU), lambda g, sidx, qidx: (qidx[0] // 32, 0)),
            ] + [sblk_spec(j) for j in range(_TCB)],
            out_specs=pl.BlockSpec((_KPAD // _TCB, _TCB),
                                   lambda g, sidx, qidx: (0, 0)),
        ),
        out_shape=jax.ShapeDtypeStruct((_KPAD // _TCB, _TCB), jnp.float32),
    )


def kernel(query, support_set, embed_table, user_sets):
    cat_embeds = _build_sc_kernel()(query, support_set, embed_table.T)
    u4 = user_sets.astype(jnp.int4)
    iou = _build_tc_kernel()(
        support_set, query, *([u4] * (_TCB + 1)))
    return cat_embeds, iou.reshape(_K, 1)


# speedup vs baseline: 6.3160x; 1.0107x over previous
import functools

import jax
import jax.numpy as jnp
from jax import lax
from jax.experimental import pallas as pl
from jax.experimental.pallas import tpu as pltpu
from jax.experimental.pallas import tpu_sc as plsc

_NUM_MOVIES = 100000
_D = 64
_NU = 512
_K = 200
_BPW = 8
_NWORK = _K // _BPW
_NS = 16
_TCB = 20
_KPAD = _K


def _sc_body(q_hbm, idx_hbm, tabT_hbm, out_e_hbm,
             idx_v, slab_v, cat_v, sem):
    wid = lax.axis_index("c") * _NS + lax.axis_index("s")

    @pl.when(wid < _NWORK)
    def _():
        base = wid * _BPW
        pltpu.sync_copy(idx_hbm.at[pl.ds(base, _BPW)], idx_v.at[pl.ds(0, _BPW)])
        pltpu.sync_copy(q_hbm, idx_v.at[pl.ds(_BPW, 1)])
        ivec = idx_v[...]

        copies = []
        for j in range(_BPW + 1):
            m = ivec[j]
            start = pl.multiple_of((m // 128) * 128, 128)
            copies.append(pltpu.make_async_copy(
                tabT_hbm.at[:, pl.ds(start, 128)], slab_v.at[j], sem))
        for c in copies:
            c.start()
        for c in copies:
            c.wait()

        lane = lax.iota(jnp.int32, 16)
        offs = ivec % 128
        qoff = jnp.full((16,), offs[_BPW], jnp.int32)
        for c in range(_D // 16):
            qchunk = plsc.load_gather(
                slab_v, [jnp.full((16,), _BPW, jnp.int32), lane + c * 16,
                         qoff])
            for j in range(_BPW):
                joff = jnp.full((16,), offs[j], jnp.int32)
                cat_v[j, pl.ds(c * 16, 16)] = plsc.load_gather(
                    slab_v, [jnp.full((16,), j, jnp.int32), lane + c * 16,
                             joff])
                cat_v[j, pl.ds(_D + c * 16, 16)] = qchunk

        pltpu.sync_copy(cat_v, out_e_hbm.at[pl.ds(base, _BPW)])


@functools.lru_cache(maxsize=None)
def _build_sc_kernel():
    return pl.kernel(
        _sc_body,
        out_type=jax.ShapeDtypeStruct((_K, 2 * _D), jnp.float32),
        mesh=plsc.VectorSubcoreMesh(core_axis_name="c", subcore_axis_name="s"),
        compiler_params=pltpu.CompilerParams(needs_layout_passes=False),
        scratch_types=[
            pltpu.VMEM((16,), jnp.int32),
            pltpu.VMEM((_BPW + 1, _D, 128), jnp.float32),
            pltpu.VMEM((_BPW, 2 * _D), jnp.float32),
            pltpu.SemaphoreType.DMA,
        ],
    )


def _tc_iou_body(sidx_ref, qidx_ref, qblk_ref, *args):
    sblk_refs = args[:_TCB]
    out_ref = args[_TCB]
    g = pl.program_id(0)
    nb = _TCB + 1

    oh_q = (lax.broadcasted_iota(jnp.int32, (1, 32), 1)
            == qidx_ref[0] % 32).astype(jnp.int4)
    qrow = jax.lax.dot_general(
        oh_q, qblk_ref[...], (((1,), (0,)), ((), ())),
        preferred_element_type=jnp.int32).astype(jnp.int4)
    v_mat = jnp.concatenate(
        [qrow, jnp.ones((1, _NU), jnp.int4)], axis=0)

    s_all = jnp.concatenate(
        [qblk_ref[...]] + [sblk_refs[j][...] for j in range(_TCB)],
        axis=0)
    p_all = jax.lax.dot_general(
        v_mat, s_all, (((1,), (1,)), ((), ())),
        preferred_element_type=jnp.int32).astype(jnp.float32)

    cols = [jnp.full((1, 1), qidx_ref[0] % 32, jnp.int32)]
    for j in range(_TCB):
        cols.append(jnp.full((1, 1),
                             32 * (j + 1) + sidx_ref[g * _TCB + j] % 32,
                             jnp.int32))
    colv = jnp.concatenate(cols, axis=0)
    oh = (lax.broadcasted_iota(jnp.int32, (nb, 32 * nb), 1)
          == colv).astype(jnp.float32)
    r = jax.lax.dot_general(
        p_all, oh, (((1,), (1,)), ((), ())),
        preferred_element_type=jnp.float32)
    inter = r[0:1, 1:]
    s_len = r[1:2, 1:]
    q_len = r[1:2, 0:1]
    union = q_len + s_len - inter
    out_ref[pl.ds(g, 1), :] = jnp.where(
        union > 0, inter / jnp.maximum(union, 1.0), 0.0)


@functools.lru_cache(maxsize=None)
def _build_tc_kernel():
    def sblk_spec(j):
        return pl.BlockSpec(
            (32, _NU), lambda g, sidx, qidx, j=j: (sidx[g * _TCB + j] // 32, 0))

    return pl.pallas_call(
        _tc_iou_body,
        grid_spec=pltpu.PrefetchScalarGridSpec(
            num_scalar_prefetch=2,
            grid=(_KPAD // _TCB,),
            in_specs=[
                pl.BlockSpec((32, _NU), lambda g, sidx, qidx: (qidx[0] // 32, 0)),
            ] + [sblk_spec(j) for j in range(_TCB)],
            out_specs=pl.BlockSpec((_KPAD // _TCB, _TCB),
                                   lambda g, sidx, qidx: (0, 0)),
        ),
        out_shape=jax.ShapeDtypeStruct((_KPAD // _TCB, _TCB), jnp.float32),
    )


def kernel(query, support_set, embed_table, user_sets):
    cat_embeds = _build_sc_kernel()(query, support_set, embed_table.T)
    u4 = user_sets.astype(jnp.int4)
    iou = _build_tc_kernel()(
        support_set, query, *([u4] * (_TCB + 1)))
    return cat_embeds, iou.reshape(_K, 1)
